# Initial kernel scaffold; baseline (speedup 1.0000x reference)
import jax
import jax.numpy as jnp
from jax.experimental import pallas as pl


def kernel(x, edge_index, batch, Wl, Wr, b, gn_w, gn_b, gn_a):
    raise NotImplementedError("write your pallas kernel here")


# speedup vs baseline: 1.0005x; 1.0005x over previous
import jax
import jax.numpy as jnp
from jax.experimental import pallas as pl

N = 10000
E = 320000
D = 128
L = 3
G = 8
NEG_SLOPE = 0.01
EPS = 1e-5
RESIDUAL_START = 1


def _div_kernel(p_ref, c_ref, o_ref):
    o_ref[...] = p_ref[...] / c_ref[...]


def kernel(x, edge_index, batch, Wl, Wr, b, gn_w, gn_b, gn_a):
    src, dst = edge_index[0], edge_index[1]
    indeg = jax.ops.segment_sum(jnp.ones(E, x.dtype), dst, num_segments=N)
    outdeg = jax.ops.segment_sum(jnp.ones(E, x.dtype), src, num_segments=N)
    inv_in = 1.0 / jnp.maximum(indeg, 1.0)
    inv_out = 1.0 / jnp.maximum(outdeg, 1.0)
    cnt_g = jax.ops.segment_sum(jnp.ones(N, x.dtype), batch, num_segments=G)
    cg = jnp.maximum(cnt_g, 1.0)

    h = x
    for i in range(L):
        aggf = jax.ops.segment_sum(h[src], dst, num_segments=N)
        aggb = jax.ops.segment_sum(h[dst], src, num_segments=N)
        m = 0.5 * (aggf * inv_in[:, None] + aggb * inv_out[:, None])
        xn = m @ Wl[i] + h @ Wr[i] + b[i]
        S1 = jax.ops.segment_sum(xn, batch, num_segments=G)
        S2 = jax.ops.segment_sum(xn * xn, batch, num_segments=G)
        mean = S1 / cg[:, None]
        alpha = gn_a[i]
        var = S2 / cg[:, None] - (2 * alpha - alpha * alpha) * mean * mean
        std = jnp.sqrt(var + EPS)
        y = gn_w[i] * (xn - alpha * mean[batch]) / std[batch] + gn_b[i]
        y = jax.nn.leaky_relu(y, NEG_SLOPE)
        h = y if i < RESIDUAL_START else h + y

    pooled = jax.ops.segment_sum(h, batch, num_segments=G)
    return pl.pallas_call(
        _div_kernel,
        out_shape=jax.ShapeDtypeStruct((G, D), x.dtype),
    )(pooled, jnp.broadcast_to(cg[:, None], (G, D)))


# speedup vs baseline: 3.4950x; 3.4933x over previous
import functools

import jax
import jax.numpy as jnp
from jax import lax
from jax.experimental import pallas as pl
from jax.experimental.pallas import tpu as pltpu
from jax.experimental.pallas import tpu_sc as plsc

N = 10000
E = 320000
D = 128
L = 3
G = 8
NEG_SLOPE = 0.01
EPS = 1e-5
RESIDUAL_START = 1

NC = 2
NS = 16
STRIPE = 632
NPAD = NS * STRIPE
TRASH = N
CH = 128
EPT = E // NS
NCHUNK = (EPT + CH - 1) // CH
EPADT = NCHUNK * CH
R = 1000
NB = N // R

_mesh = plsc.VectorSubcoreMesh(
    core_axis_name="c", subcore_axis_name="s", num_cores=NC, num_subcores=NS)


@functools.partial(
    pl.kernel,
    out_type=jax.ShapeDtypeStruct((NC, NPAD, D), jnp.float32),
    mesh=_mesh,
    scratch_types=[
        pltpu.VMEM_SHARED((NPAD, D), jnp.float32),
        pltpu.VMEM((CH,), jnp.int32),
        pltpu.VMEM((CH,), jnp.int32),
        pltpu.VMEM((CH, D), jnp.float32),
        pltpu.SemaphoreType.DMA,
    ],
)
def _agg_kernel(h_hbm, edges_hbm, zeros_hbm, agg_hbm,
                acc_s, gidx_v, sidx_v, buf_v, gsem):
    c = lax.axis_index("c")
    s = lax.axis_index("s")
    rows = pl.ds(s * STRIPE, STRIPE)
    pltpu.sync_copy(zeros_hbm, acc_s.at[rows])
    plsc.subcore_barrier()

    def body(j, carry):
        pltpu.sync_copy(edges_hbm.at[c, s, j], gidx_v)
        pltpu.sync_copy(edges_hbm.at[1 - c, s, j], sidx_v)
        pltpu.async_copy(h_hbm.at[gidx_v], buf_v, gsem).wait()
        pltpu.sync_copy(buf_v, acc_s.at[sidx_v], add=True)
        return carry

    lax.fori_loop(0, NCHUNK, body, 0)
    plsc.subcore_barrier()
    pltpu.sync_copy(acc_s.at[rows], agg_hbm.at[c, rows])


def _mm_stats_body(h_b, af_b, ab_b, df_b, db_b, b3_b, wl_r, wr_r, bias_r,
                   xn_o, s1_o, s2_o, cnt_o, s1_s, s2_s, cnt_s):
    i = pl.program_id(0)

    @pl.when(i == 0)
    def _init():
        s1_s[...] = jnp.zeros_like(s1_s)
        s2_s[...] = jnp.zeros_like(s2_s)
        cnt_s[...] = jnp.zeros_like(cnt_s)

    af = af_b[0]
    ab = ab_b[0]
    m = 0.5 * (af / jnp.maximum(df_b[0], 1.0) + ab / jnp.maximum(db_b[0], 1.0))
    xn = (jnp.dot(m, wl_r[...], preferred_element_type=jnp.float32)
          + jnp.dot(h_b[...], wr_r[...], preferred_element_type=jnp.float32)
          + bias_r[...])
    xn_o[...] = xn

    bvec = b3_b[0, 0, :]
    onehot_t = (lax.broadcasted_iota(jnp.int32, (G, R), 0)
                == bvec[None, :]).astype(jnp.float32)
    s1_s[...] += lax.dot_general(onehot_t, xn, (((1,), (0,)), ((), ())),
                                 preferred_element_type=jnp.float32)
    s2_s[...] += lax.dot_general(onehot_t, xn * xn, (((1,), (0,)), ((), ())),
                                 preferred_element_type=jnp.float32)
    cnt_s[...] += jnp.broadcast_to(jnp.sum(onehot_t, axis=1)[:, None], (G, D))

    @pl.when(i == NB - 1)
    def _fin():
        s1_o[...] = s1_s[...]
        s2_o[...] = s2_s[...]
        cnt_o[...] = cnt_s[...]


def _mm_stats(h, agg, deg, b3, wl, wr, bias):
    blk = lambda i: (i, 0)
    rep = lambda i: (0, 0)
    fwd = lambda i: (0, i, 0)
    bwd = lambda i: (1, i, 0)
    return pl.pallas_call(
        _mm_stats_body,
        grid=(NB,),
        in_specs=[
            pl.BlockSpec((R, D), blk),
            pl.BlockSpec((1, R, D), fwd),
            pl.BlockSpec((1, R, D), bwd),
            pl.BlockSpec((1, R, D), fwd),
            pl.BlockSpec((1, R, D), bwd),
            pl.BlockSpec((1, 1, R), lambda i: (i, 0, 0)),
            pl.BlockSpec((D, D), rep),
            pl.BlockSpec((D, D), rep),
            pl.BlockSpec((1, D), rep),
        ],
        out_specs=[
            pl.BlockSpec((R, D), blk),
            pl.BlockSpec((G, D), rep),
            pl.BlockSpec((G, D), rep),
            pl.BlockSpec((G, D), rep),
        ],
        out_shape=[
            jax.ShapeDtypeStruct((N, D), jnp.float32),
            jax.ShapeDtypeStruct((G, D), jnp.float32),
            jax.ShapeDtypeStruct((G, D), jnp.float32),
            jax.ShapeDtypeStruct((G, D), jnp.float32),
        ],
        scratch_shapes=[pltpu.VMEM((G, D), jnp.float32)] * 3,
    )(h, agg, agg, deg, deg, b3, wl, wr, bias)


def _norm_body(residual, pool, xn_b, hp_b, b3_b, s1_r, s2_r, cnt_r, w_r, bb_r,
               a_r, out_o, *scratch):
    i = pl.program_id(0)
    cg = jnp.maximum(cnt_r[...], 1.0)
    mean = s1_r[...] / cg
    alpha = a_r[...]
    var = s2_r[...] / cg - (2.0 * alpha - alpha * alpha) * mean * mean
    inv_std = lax.rsqrt(var + EPS)

    bvec = b3_b[0, 0, :]
    onehot_t = (lax.broadcasted_iota(jnp.int32, (G, R), 0)
                == bvec[None, :]).astype(jnp.float32)
    mean_b = lax.dot_general(onehot_t, mean, (((0,), (0,)), ((), ())),
                             preferred_element_type=jnp.float32)
    istd_b = lax.dot_general(onehot_t, inv_std, (((0,), (0,)), ((), ())),
                             preferred_element_type=jnp.float32)
    y = w_r[...] * (xn_b[...] - alpha * mean_b) * istd_b + bb_r[...]
    y = jnp.where(y >= 0.0, y, NEG_SLOPE * y)
    if residual:
        y = y + hp_b[...]

    if pool:
        pool_s = scratch[0]

        @pl.when(i == 0)
        def _init():
            pool_s[...] = jnp.zeros_like(pool_s)

        pool_s[...] += lax.dot_general(onehot_t, y, (((1,), (0,)), ((), ())),
                                       preferred_element_type=jnp.float32)

        @pl.when(i == NB - 1)
        def _fin():
            out_o[...] = pool_s[...] / cg
    else:
        out_o[...] = y


def _norm_apply(xn, hprev, b3, s1, s2, cnt, w, bb, a, residual, pool):
    blk = lambda i: (i, 0)
    rep = lambda i: (0, 0)
    if pool:
        out_spec = pl.BlockSpec((G, D), rep)
        out_shape = jax.ShapeDtypeStruct((G, D), jnp.float32)
        scratch = [pltpu.VMEM((G, D), jnp.float32)]
    else:
        out_spec = pl.BlockSpec((R, D), blk)
        out_shape = jax.ShapeDtypeStruct((N, D), jnp.float32)
        scratch = []
    return pl.pallas_call(
        functools.partial(_norm_body, residual, pool),
        grid=(NB,),
        in_specs=[
            pl.BlockSpec((R, D), blk),
            pl.BlockSpec((R, D), blk),
            pl.BlockSpec((1, 1, R), lambda i: (i, 0, 0)),
            pl.BlockSpec((G, D), rep),
            pl.BlockSpec((G, D), rep),
            pl.BlockSpec((G, D), rep),
            pl.BlockSpec((1, D), rep),
            pl.BlockSpec((1, D), rep),
            pl.BlockSpec((1, D), rep),
        ],
        out_specs=out_spec,
        out_shape=out_shape,
        scratch_shapes=scratch,
    )(xn, hprev, b3, s1, s2, cnt, w, bb, a)


def kernel(x, edge_index, batch, Wl, Wr, b, gn_w, gn_b, gn_a):
    f32 = jnp.float32
    src = edge_index[0].astype(jnp.int32)
    dst = edge_index[1].astype(jnp.int32)
    pad_e = NS * EPADT - E
    srcp = jnp.concatenate(
        [src, jnp.full((pad_e,), TRASH, jnp.int32)]).reshape(NS, NCHUNK, CH)
    dstp = jnp.concatenate(
        [dst, jnp.full((pad_e,), TRASH, jnp.int32)]).reshape(NS, NCHUNK, CH)
    edges = jnp.stack([srcp, dstp])

    zeros = jnp.zeros((STRIPE, D), f32)
    ones = jnp.ones((NPAD, D), f32)

    deg = _agg_kernel(ones, edges, zeros)
    batch3 = batch.astype(jnp.int32).reshape(NB, 1, R)

    h = x
    out = None
    for i in range(L):
        hp = jnp.pad(h, ((0, NPAD - N), (0, 0)))
        agg = _agg_kernel(hp, edges, zeros)
        xn, s1, s2, cnt = _mm_stats(h, agg, deg, batch3,
                                    Wl[i], Wr[i], b[i].reshape(1, D))
        last = i == L - 1
        res = i >= RESIDUAL_START
        nxt = _norm_apply(xn, h, batch3, s1, s2, cnt,
                          gn_w[i].reshape(1, D), gn_b[i].reshape(1, D),
                          gn_a[i].reshape(1, D), residual=res, pool=last)
        if last:
            out = nxt
        else:
            h = nxt
    return out


# speedup vs baseline: 3.9506x; 1.1304x over previous
import functools

import jax
import jax.numpy as jnp
from jax import lax
from jax.experimental import pallas as pl
from jax.experimental.pallas import tpu as pltpu
from jax.experimental.pallas import tpu_sc as plsc

N = 10000
E = 320000
D = 128
L = 3
G = 8
NEG_SLOPE = 0.01
EPS = 1e-5
RESIDUAL_START = 1

NC = 2
NS = 16
STRIPE = 632
NPAD = NS * STRIPE
TRASH = N
CH = 128
NCHUNK = 158
EPADT = NCHUNK * CH
R = 1000
NB = N // R

_mesh = plsc.VectorSubcoreMesh(
    core_axis_name="c", subcore_axis_name="s", num_cores=NC, num_subcores=NS)


@functools.partial(
    pl.kernel,
    out_type=jax.ShapeDtypeStruct((NC, NPAD, D), jnp.float32),
    mesh=_mesh,
    scratch_types=[
        pltpu.VMEM_SHARED((NPAD, D), jnp.float32),
        pltpu.VMEM((2, 2, CH), jnp.int32),
        pltpu.VMEM((2, CH, D), jnp.float32),
        pltpu.SemaphoreType.DMA,
        pltpu.SemaphoreType.DMA,
    ],
)
def _agg_kernel(h_hbm, edges_hbm, zeros_hbm, agg_hbm,
                acc_s, pair_v, buf_v, gsem0, gsem1):
    c = lax.axis_index("c")
    s = lax.axis_index("s")
    rows = pl.ds(s * STRIPE, STRIPE)
    pltpu.sync_copy(zeros_hbm, acc_s.at[rows])
    plsc.subcore_barrier()

    pltpu.sync_copy(edges_hbm.at[s, 0], pair_v.at[0])
    pltpu.async_copy(h_hbm.at[pair_v.at[0, c]], buf_v.at[0], gsem0)

    def loop(jj, carry):
        c1 = 2 * jj + 1
        c2 = 2 * jj + 2
        pltpu.sync_copy(edges_hbm.at[s, c1], pair_v.at[1])
        pltpu.async_copy(h_hbm.at[pair_v.at[1, c]], buf_v.at[1], gsem1)
        pltpu.make_async_copy(
            h_hbm.at[pair_v.at[0, c]], buf_v.at[0], gsem0).wait()
        pltpu.sync_copy(buf_v.at[0], acc_s.at[pair_v.at[0, 1 - c]], add=True)

        @pl.when(c2 < NCHUNK)
        def _():
            pltpu.sync_copy(edges_hbm.at[s, c2], pair_v.at[0])
            pltpu.async_copy(h_hbm.at[pair_v.at[0, c]], buf_v.at[0], gsem0)

        pltpu.make_async_copy(
            h_hbm.at[pair_v.at[1, c]], buf_v.at[1], gsem1).wait()
        pltpu.sync_copy(buf_v.at[1], acc_s.at[pair_v.at[1, 1 - c]], add=True)
        return carry

    lax.fori_loop(0, NCHUNK // 2, loop, 0)
    plsc.subcore_barrier()
    pltpu.sync_copy(acc_s.at[rows], agg_hbm.at[c, rows])


@functools.partial(
    pl.kernel,
    out_type=jax.ShapeDtypeStruct((NC, NPAD, 16), jnp.float32),
    mesh=_mesh,
    scratch_types=[
        pltpu.VMEM_SHARED((NPAD, 16), jnp.float32),
        pltpu.VMEM((CH,), jnp.int32),
        pltpu.VMEM((CH, 16), jnp.float32),
    ],
)
def _deg_kernel(edges_hbm, zeros16_hbm, ones16_hbm, deg_hbm,
                dacc_s, idx_v, ones_v):
    c = lax.axis_index("c")
    s = lax.axis_index("s")
    rows = pl.ds(s * STRIPE, STRIPE)
    pltpu.sync_copy(zeros16_hbm, dacc_s.at[rows])
    pltpu.sync_copy(ones16_hbm, ones_v)
    plsc.subcore_barrier()

    def loop(j, carry):
        pltpu.sync_copy(edges_hbm.at[s, j, 1 - c], idx_v)
        pltpu.sync_copy(ones_v, dacc_s.at[idx_v], add=True)
        return carry

    lax.fori_loop(0, NCHUNK, loop, 0)
    plsc.subcore_barrier()
    pltpu.sync_copy(dacc_s.at[rows], deg_hbm.at[c, rows])


def _mm_stats_body(h_b, af_b, ab_b, ii_b, io_b, b3_b, wl_r, wr_r, bias_r,
                   xn_o, s1_o, s2_o, cnt_o, s1_s, s2_s, cnt_s):
    i = pl.program_id(0)

    @pl.when(i == 0)
    def _init():
        s1_s[...] = jnp.zeros_like(s1_s)
        s2_s[...] = jnp.zeros_like(s2_s)
        cnt_s[...] = jnp.zeros_like(cnt_s)

    m = 0.5 * (af_b[0] * ii_b[...] + ab_b[0] * io_b[...])
    xn = (jnp.dot(m, wl_r[...], preferred_element_type=jnp.float32)
          + jnp.dot(h_b[...], wr_r[...], preferred_element_type=jnp.float32)
          + bias_r[...])
    xn_o[...] = xn

    bvec = b3_b[0, 0, :]
    onehot_t = (lax.broadcasted_iota(jnp.int32, (G, R), 0)
                == bvec[None, :]).astype(jnp.float32)
    s1_s[...] += lax.dot_general(onehot_t, xn, (((1,), (0,)), ((), ())),
                                 preferred_element_type=jnp.float32)
    s2_s[...] += lax.dot_general(onehot_t, xn * xn, (((1,), (0,)), ((), ())),
                                 preferred_element_type=jnp.float32)
    cnt_s[...] += jnp.broadcast_to(jnp.sum(onehot_t, axis=1)[:, None], (G, D))

    @pl.when(i == NB - 1)
    def _fin():
        s1_o[...] = s1_s[...]
        s2_o[...] = s2_s[...]
        cnt_o[...] = cnt_s[...]


def _mm_stats(h, agg, ii, io, b3, wl, wr, bias):
    blk = lambda i: (i, 0)
    rep = lambda i: (0, 0)
    fwd = lambda i: (0, i, 0)
    bwd = lambda i: (1, i, 0)
    return pl.pallas_call(
        _mm_stats_body,
        grid=(NB,),
        in_specs=[
            pl.BlockSpec((R, D), blk),
            pl.BlockSpec((1, R, D), fwd),
            pl.BlockSpec((1, R, D), bwd),
            pl.BlockSpec((R, D), blk),
            pl.BlockSpec((R, D), blk),
            pl.BlockSpec((1, 1, R), lambda i: (i, 0, 0)),
            pl.BlockSpec((D, D), rep),
            pl.BlockSpec((D, D), rep),
            pl.BlockSpec((1, D), rep),
        ],
        out_specs=[
            pl.BlockSpec((R, D), blk),
            pl.BlockSpec((G, D), rep),
            pl.BlockSpec((G, D), rep),
            pl.BlockSpec((G, D), rep),
        ],
        out_shape=[
            jax.ShapeDtypeStruct((N, D), jnp.float32),
            jax.ShapeDtypeStruct((G, D), jnp.float32),
            jax.ShapeDtypeStruct((G, D), jnp.float32),
            jax.ShapeDtypeStruct((G, D), jnp.float32),
        ],
        scratch_shapes=[pltpu.VMEM((G, D), jnp.float32)] * 3,
    )(h, agg, agg, ii, io, b3, wl, wr, bias)


def _norm_body(residual, pool, xn_b, hp_b, b3_b, s1_r, s2_r, cnt_r, w_r, bb_r,
               a_r, out_o, *scratch):
    i = pl.program_id(0)
    cg = jnp.maximum(cnt_r[...], 1.0)
    mean = s1_r[...] / cg
    alpha = a_r[...]
    var = s2_r[...] / cg - (2.0 * alpha - alpha * alpha) * mean * mean
    inv_std = lax.rsqrt(var + EPS)

    bvec = b3_b[0, 0, :]
    onehot_t = (lax.broadcasted_iota(jnp.int32, (G, R), 0)
                == bvec[None, :]).astype(jnp.float32)
    mean_b = lax.dot_general(onehot_t, mean, (((0,), (0,)), ((), ())),
                             preferred_element_type=jnp.float32)
    istd_b = lax.dot_general(onehot_t, inv_std, (((0,), (0,)), ((), ())),
                             preferred_element_type=jnp.float32)
    y = w_r[...] * (xn_b[...] - alpha * mean_b) * istd_b + bb_r[...]
    y = jnp.where(y >= 0.0, y, NEG_SLOPE * y)
    if residual:
        y = y + hp_b[...]

    if pool:
        pool_s = scratch[0]

        @pl.when(i == 0)
        def _init():
            pool_s[...] = jnp.zeros_like(pool_s)

        pool_s[...] += lax.dot_general(onehot_t, y, (((1,), (0,)), ((), ())),
                                       preferred_element_type=jnp.float32)

        @pl.when(i == NB - 1)
        def _fin():
            out_o[...] = pool_s[...] / cg
    else:
        out_o[...] = y


def _norm_apply(xn, hprev, b3, s1, s2, cnt, w, bb, a, residual, pool):
    blk = lambda i: (i, 0)
    rep = lambda i: (0, 0)
    if pool:
        out_spec = pl.BlockSpec((G, D), rep)
        out_shape = jax.ShapeDtypeStruct((G, D), jnp.float32)
        scratch = [pltpu.VMEM((G, D), jnp.float32)]
    else:
        out_spec = pl.BlockSpec((R, D), blk)
        out_shape = jax.ShapeDtypeStruct((N, D), jnp.float32)
        scratch = []
    return pl.pallas_call(
        functools.partial(_norm_body, residual, pool),
        grid=(NB,),
        in_specs=[
            pl.BlockSpec((R, D), blk),
            pl.BlockSpec((R, D), blk),
            pl.BlockSpec((1, 1, R), lambda i: (i, 0, 0)),
            pl.BlockSpec((G, D), rep),
            pl.BlockSpec((G, D), rep),
            pl.BlockSpec((G, D), rep),
            pl.BlockSpec((1, D), rep),
            pl.BlockSpec((1, D), rep),
            pl.BlockSpec((1, D), rep),
        ],
        out_specs=out_spec,
        out_shape=out_shape,
        scratch_shapes=scratch,
    )(xn, hprev, b3, s1, s2, cnt, w, bb, a)


def kernel(x, edge_index, batch, Wl, Wr, b, gn_w, gn_b, gn_a):
    f32 = jnp.float32
    src = edge_index[0].astype(jnp.int32)
    dst = edge_index[1].astype(jnp.int32)
    pad_e = NS * EPADT - E
    srcp = jnp.concatenate(
        [src, jnp.full((pad_e,), TRASH, jnp.int32)]).reshape(NS, NCHUNK, CH)
    dstp = jnp.concatenate(
        [dst, jnp.full((pad_e,), TRASH, jnp.int32)]).reshape(NS, NCHUNK, CH)
    edges = jnp.stack([srcp, dstp], axis=2)

    zeros = jnp.zeros((STRIPE, D), f32)
    zeros16 = jnp.zeros((STRIPE, 16), f32)
    ones16 = jnp.ones((CH, 16), f32)

    batch3 = batch.astype(jnp.int32).reshape(NB, 1, R)

    degf = _agg_kernel(jnp.ones((NPAD, D), f32), edges, zeros)
    ii = 1.0 / jnp.maximum(degf[0, :N], 1.0)
    io = 1.0 / jnp.maximum(degf[1, :N], 1.0)

    h = x
    out = None
    for i in range(L):
        hp = jnp.pad(h, ((0, NPAD - N), (0, 0)))
        agg = _agg_kernel(hp, edges, zeros)
        xn, s1, s2, cnt = _mm_stats(h, agg, ii, io, batch3,
                                    Wl[i], Wr[i], b[i].reshape(1, D))
        last = i == L - 1
        res = i >= RESIDUAL_START
        nxt = _norm_apply(xn, h, batch3, s1, s2, cnt,
                          gn_w[i].reshape(1, D), gn_b[i].reshape(1, D),
                          gn_a[i].reshape(1, D), residual=res, pool=last)
        if last:
            out = nxt
        else:
            h = nxt
    return out


# speedup vs baseline: 4.6582x; 1.1791x over previous
import functools

import jax
import jax.numpy as jnp
from jax import lax
from jax.experimental import pallas as pl
from jax.experimental.pallas import tpu as pltpu
from jax.experimental.pallas import tpu_sc as plsc

N = 10000
E = 320000
D = 128
L = 3
G = 8
NEG_SLOPE = 0.01
EPS = 1e-5
RESIDUAL_START = 1

NC = 2
NS = 16
STRIPE = 632
NPAD = NS * STRIPE
TRASH = N
CH = 120
NCHUNK = 168
EPADT = NCHUNK * CH
R = 1000
NB = N // R

_mesh = plsc.VectorSubcoreMesh(
    core_axis_name="c", subcore_axis_name="s", num_cores=NC, num_subcores=NS)


@functools.partial(
    pl.kernel,
    out_type=jax.ShapeDtypeStruct((NC, NPAD, D), jnp.float32),
    mesh=_mesh,
    scratch_types=[
        pltpu.VMEM_SHARED((NPAD, D), jnp.float32),
        pltpu.VMEM((3, 2, CH), jnp.int32),
        pltpu.VMEM((3, CH, D), jnp.float32),
        pltpu.SemaphoreType.DMA,
        pltpu.SemaphoreType.DMA,
        pltpu.SemaphoreType.DMA,
    ],
)
def _agg_kernel(h_hbm, edges_hbm, zeros_hbm, agg_hbm,
                acc_s, pair_v, buf_v, gsem0, gsem1, gsem2):
    c = lax.axis_index("c")
    s = lax.axis_index("s")
    rows = pl.ds(s * STRIPE, STRIPE)
    sems = (gsem0, gsem1, gsem2)
    pltpu.sync_copy(zeros_hbm, acc_s.at[rows])
    plsc.subcore_barrier()

    for k in range(3):
        pltpu.sync_copy(edges_hbm.at[s, k], pair_v.at[k])
        pltpu.async_copy(h_hbm.at[pair_v.at[k, c]], buf_v.at[k], sems[k])

    def loop(g, carry):
        b = 3 * g
        for k in range(3):
            ck = b + k
            pltpu.make_async_copy(
                h_hbm.at[pair_v.at[k, c]], buf_v.at[k], sems[k]).wait()
            pltpu.sync_copy(buf_v.at[k], acc_s.at[pair_v.at[k, 1 - c]],
                            add=True)

            @pl.when(ck + 3 < NCHUNK)
            def _():
                pltpu.sync_copy(edges_hbm.at[s, ck + 3], pair_v.at[k])
                pltpu.async_copy(h_hbm.at[pair_v.at[k, c]], buf_v.at[k],
                                 sems[k])
        return carry

    lax.fori_loop(0, NCHUNK // 3, loop, 0)
    plsc.subcore_barrier()
    pltpu.sync_copy(acc_s.at[rows], agg_hbm.at[c, rows])


def _mm_stats_body(h_b, af_b, ab_b, ii_b, io_b, b3_b, wl_r, wr_r, bias_r,
                   xn_o, s1_o, s2_o, cnt_o, s1_s, s2_s, cnt_s):
    i = pl.program_id(0)

    @pl.when(i == 0)
    def _init():
        s1_s[...] = jnp.zeros_like(s1_s)
        s2_s[...] = jnp.zeros_like(s2_s)
        cnt_s[...] = jnp.zeros_like(cnt_s)

    m = 0.5 * (af_b[0] * ii_b[...] + ab_b[0] * io_b[...])
    xn = (jnp.dot(m, wl_r[...], preferred_element_type=jnp.float32)
          + jnp.dot(h_b[...], wr_r[...], preferred_element_type=jnp.float32)
          + bias_r[...])
    xn_o[...] = xn

    bvec = b3_b[0, 0, :]
    onehot_t = (lax.broadcasted_iota(jnp.int32, (G, R), 0)
                == bvec[None, :]).astype(jnp.float32)
    s1_s[...] += lax.dot_general(onehot_t, xn, (((1,), (0,)), ((), ())),
                                 preferred_element_type=jnp.float32)
    s2_s[...] += lax.dot_general(onehot_t, xn * xn, (((1,), (0,)), ((), ())),
                                 preferred_element_type=jnp.float32)
    cnt_s[...] += jnp.broadcast_to(jnp.sum(onehot_t, axis=1)[:, None], (G, D))

    @pl.when(i == NB - 1)
    def _fin():
        s1_o[...] = s1_s[...]
        s2_o[...] = s2_s[...]
        cnt_o[...] = cnt_s[...]


def _mm_stats(h, agg, ii, io, b3, wl, wr, bias):
    blk = lambda i: (i, 0)
    rep = lambda i: (0, 0)
    fwd = lambda i: (0, i, 0)
    bwd = lambda i: (1, i, 0)
    return pl.pallas_call(
        _mm_stats_body,
        grid=(NB,),
        in_specs=[
            pl.BlockSpec((R, D), blk),
            pl.BlockSpec((1, R, D), fwd),
            pl.BlockSpec((1, R, D), bwd),
            pl.BlockSpec((R, D), blk),
            pl.BlockSpec((R, D), blk),
            pl.BlockSpec((1, 1, R), lambda i: (i, 0, 0)),
            pl.BlockSpec((D, D), rep),
            pl.BlockSpec((D, D), rep),
            pl.BlockSpec((1, D), rep),
        ],
        out_specs=[
            pl.BlockSpec((R, D), blk),
            pl.BlockSpec((G, D), rep),
            pl.BlockSpec((G, D), rep),
            pl.BlockSpec((G, D), rep),
        ],
        out_shape=[
            jax.ShapeDtypeStruct((N, D), jnp.float32),
            jax.ShapeDtypeStruct((G, D), jnp.float32),
            jax.ShapeDtypeStruct((G, D), jnp.float32),
            jax.ShapeDtypeStruct((G, D), jnp.float32),
        ],
        scratch_shapes=[pltpu.VMEM((G, D), jnp.float32)] * 3,
    )(h, agg, agg, ii, io, b3, wl, wr, bias)


def _norm_body(residual, pool, xn_b, hp_b, b3_b, s1_r, s2_r, cnt_r, w_r, bb_r,
               a_r, out_o, *scratch):
    i = pl.program_id(0)
    cg = jnp.maximum(cnt_r[...], 1.0)
    mean = s1_r[...] / cg
    alpha = a_r[...]
    var = s2_r[...] / cg - (2.0 * alpha - alpha * alpha) * mean * mean
    inv_std = lax.rsqrt(var + EPS)

    bvec = b3_b[0, 0, :]
    onehot_t = (lax.broadcasted_iota(jnp.int32, (G, R), 0)
                == bvec[None, :]).astype(jnp.float32)
    mean_b = lax.dot_general(onehot_t, mean, (((0,), (0,)), ((), ())),
                             preferred_element_type=jnp.float32)
    istd_b = lax.dot_general(onehot_t, inv_std, (((0,), (0,)), ((), ())),
                             preferred_element_type=jnp.float32)
    y = w_r[...] * (xn_b[...] - alpha * mean_b) * istd_b + bb_r[...]
    y = jnp.where(y >= 0.0, y, NEG_SLOPE * y)
    if residual:
        y = y + hp_b[...]

    if pool:
        pool_s = scratch[0]

        @pl.when(i == 0)
        def _init():
            pool_s[...] = jnp.zeros_like(pool_s)

        pool_s[...] += lax.dot_general(onehot_t, y, (((1,), (0,)), ((), ())),
                                       preferred_element_type=jnp.float32)

        @pl.when(i == NB - 1)
        def _fin():
            out_o[...] = pool_s[...] / cg
    else:
        out_o[...] = y


def _norm_apply(xn, hprev, b3, s1, s2, cnt, w, bb, a, residual, pool):
    blk = lambda i: (i, 0)
    rep = lambda i: (0, 0)
    if pool:
        out_spec = pl.BlockSpec((G, D), rep)
        out_shape = jax.ShapeDtypeStruct((G, D), jnp.float32)
        scratch = [pltpu.VMEM((G, D), jnp.float32)]
    else:
        out_spec = pl.BlockSpec((R, D), blk)
        out_shape = jax.ShapeDtypeStruct((N, D), jnp.float32)
        scratch = []
    return pl.pallas_call(
        functools.partial(_norm_body, residual, pool),
        grid=(NB,),
        in_specs=[
            pl.BlockSpec((R, D), blk),
            pl.BlockSpec((R, D), blk),
            pl.BlockSpec((1, 1, R), lambda i: (i, 0, 0)),
            pl.BlockSpec((G, D), rep),
            pl.BlockSpec((G, D), rep),
            pl.BlockSpec((G, D), rep),
            pl.BlockSpec((1, D), rep),
            pl.BlockSpec((1, D), rep),
            pl.BlockSpec((1, D), rep),
        ],
        out_specs=out_spec,
        out_shape=out_shape,
        scratch_shapes=scratch,
    )(xn, hprev, b3, s1, s2, cnt, w, bb, a)


def kernel(x, edge_index, batch, Wl, Wr, b, gn_w, gn_b, gn_a):
    f32 = jnp.float32
    src = edge_index[0].astype(jnp.int32)
    dst = edge_index[1].astype(jnp.int32)
    pad_e = NS * EPADT - E
    srcp = jnp.concatenate(
        [src, jnp.full((pad_e,), TRASH, jnp.int32)]).reshape(NS, NCHUNK, CH)
    dstp = jnp.concatenate(
        [dst, jnp.full((pad_e,), TRASH, jnp.int32)]).reshape(NS, NCHUNK, CH)
    edges = jnp.stack([srcp, dstp], axis=2)

    zeros = jnp.zeros((STRIPE, D), f32)

    batch3 = batch.astype(jnp.int32).reshape(NB, 1, R)

    degf = _agg_kernel(jnp.ones((NPAD, D), f32), edges, zeros)
    ii = 1.0 / jnp.maximum(degf[0, :N], 1.0)
    io = 1.0 / jnp.maximum(degf[1, :N], 1.0)

    h = x
    out = None
    for i in range(L):
        hp = jnp.pad(h, ((0, NPAD - N), (0, 0)))
        agg = _agg_kernel(hp, edges, zeros)
        xn, s1, s2, cnt = _mm_stats(h, agg, ii, io, batch3,
                                    Wl[i], Wr[i], b[i].reshape(1, D))
        last = i == L - 1
        res = i >= RESIDUAL_START
        nxt = _norm_apply(xn, h, batch3, s1, s2, cnt,
                          gn_w[i].reshape(1, D), gn_b[i].reshape(1, D),
                          gn_a[i].reshape(1, D), residual=res, pool=last)
        if last:
            out = nxt
        else:
            h = nxt
    return out


# speedup vs baseline: 5.5272x; 1.1866x over previous
import functools

import jax
import jax.numpy as jnp
from jax import lax
from jax.experimental import pallas as pl
from jax.experimental.pallas import tpu as pltpu
from jax.experimental.pallas import tpu_sc as plsc

N = 10000
E = 320000
D = 128
L = 3
G = 8
NEG_SLOPE = 0.01
EPS = 1e-5
RESIDUAL_START = 1

NC = 2
NS = 16
STRIPE = 632
NPAD = NS * STRIPE
TRASH = N
CH = 112
NCHUNK = 180
NCPAD = NCHUNK + 3
EPADT = NCHUNK * CH
R = 1000
NB = N // R

_mesh = plsc.VectorSubcoreMesh(
    core_axis_name="c", subcore_axis_name="s", num_cores=NC, num_subcores=NS)


@functools.partial(
    pl.kernel,
    out_type=jax.ShapeDtypeStruct((NC, NPAD, D), jnp.float32),
    mesh=_mesh,
    scratch_types=[
        pltpu.VMEM_SHARED((NPAD, D), jnp.float32),
        pltpu.VMEM((2, 3, 2, CH), jnp.int32),
        pltpu.VMEM((3, CH, D), jnp.float32),
        pltpu.SemaphoreType.DMA,
        pltpu.SemaphoreType.DMA,
        pltpu.SemaphoreType.DMA,
        pltpu.SemaphoreType.DMA,
        pltpu.SemaphoreType.DMA,
        pltpu.SemaphoreType.DMA,
    ],
)
def _agg_kernel(h_hbm, edges_hbm, zeros_hbm, agg_hbm,
                acc_s, pblk_v, buf_v, g0, g1, g2, s0, s1, s2):
    c = lax.axis_index("c")
    s = lax.axis_index("s")
    rows = pl.ds(s * STRIPE, STRIPE)
    gsems = (g0, g1, g2)
    ssems = (s0, s1, s2)
    pltpu.sync_copy(zeros_hbm, acc_s.at[rows])
    plsc.subcore_barrier()

    pltpu.sync_copy(edges_hbm.at[s, pl.ds(0, 3)], pblk_v.at[1])
    for k in range(3):
        pltpu.async_copy(h_hbm.at[pblk_v.at[1, k, c]], buf_v.at[k], gsems[k])

    def loop(g, carry):
        b = 3 * g
        p = lax.rem(g, 2)
        pltpu.sync_copy(edges_hbm.at[s, pl.ds(b + 3, 3)], pblk_v.at[p])
        for k in range(3):
            pltpu.make_async_copy(
                h_hbm.at[pblk_v.at[1 - p, k, c]], buf_v.at[k],
                gsems[k]).wait()
            pltpu.async_copy(buf_v.at[k], acc_s.at[pblk_v.at[1 - p, k, 1 - c]],
                             ssems[k], add=True)
        for k in range(3):
            ck = b + k

            @pl.when(ck + 3 < NCHUNK)
            def _():
                pltpu.make_async_copy(
                    buf_v.at[k], acc_s.at[pblk_v.at[1 - p, k, 1 - c]],
                    ssems[k]).wait()
                pltpu.async_copy(h_hbm.at[pblk_v.at[p, k, c]], buf_v.at[k],
                                 gsems[k])
        return carry

    lax.fori_loop(0, NCHUNK // 3, loop, 0)
    for k in range(3):
        pltpu.make_async_copy(
            buf_v.at[k], acc_s.at[pblk_v.at[0, k, 1]], ssems[k]).wait()
    plsc.subcore_barrier()
    pltpu.sync_copy(acc_s.at[rows], agg_hbm.at[c, rows])


@functools.partial(
    pl.kernel,
    out_type=jax.ShapeDtypeStruct((NC, NPAD, D), jnp.float32),
    mesh=_mesh,
    scratch_types=[
        pltpu.VMEM_SHARED((NPAD, D), jnp.float32),
        pltpu.VMEM((2, 3, 2, CH), jnp.int32),
        pltpu.VMEM((CH, D), jnp.float32),
        pltpu.SemaphoreType.DMA,
        pltpu.SemaphoreType.DMA,
        pltpu.SemaphoreType.DMA,
    ],
)
def _deg_kernel(edges_hbm, zeros_hbm, ones_hbm, deg_hbm,
                dacc_s, pblk_v, ones_v, s0, s1, s2):
    c = lax.axis_index("c")
    s = lax.axis_index("s")
    rows = pl.ds(s * STRIPE, STRIPE)
    ssems = (s0, s1, s2)
    pltpu.sync_copy(zeros_hbm, dacc_s.at[rows])
    pltpu.sync_copy(ones_hbm, ones_v)
    plsc.subcore_barrier()

    pltpu.sync_copy(edges_hbm.at[s, pl.ds(0, 3)], pblk_v.at[1])
    for k in range(3):
        pltpu.async_copy(ones_v, dacc_s.at[pblk_v.at[1, k, 1 - c]], ssems[k],
                         add=True)

    def loop(g, carry):
        b = 3 * g
        p = lax.rem(g, 2)
        pltpu.sync_copy(edges_hbm.at[s, pl.ds(b + 3, 3)], pblk_v.at[p])
        for k in range(3):
            ck = b + k
            pltpu.make_async_copy(
                ones_v, dacc_s.at[pblk_v.at[1 - p, k, 1 - c]],
                ssems[k]).wait()

            @pl.when(ck + 3 < NCHUNK)
            def _():
                pltpu.async_copy(ones_v, dacc_s.at[pblk_v.at[p, k, 1 - c]],
                                 ssems[k], add=True)
        return carry

    lax.fori_loop(0, NCHUNK // 3, loop, 0)
    plsc.subcore_barrier()
    pltpu.sync_copy(dacc_s.at[rows], deg_hbm.at[c, rows])


def _mm_stats_body(h_b, af_b, ab_b, ii_b, io_b, b3_b, wl_r, wr_r, bias_r,
                   xn_o, s1_o, s2_o, cnt_o, s1_s, s2_s, cnt_s):
    i = pl.program_id(0)

    @pl.when(i == 0)
    def _init():
        s1_s[...] = jnp.zeros_like(s1_s)
        s2_s[...] = jnp.zeros_like(s2_s)
        cnt_s[...] = jnp.zeros_like(cnt_s)

    m = 0.5 * (af_b[0] * ii_b[...] + ab_b[0] * io_b[...])
    xn = (jnp.dot(m, wl_r[...], preferred_element_type=jnp.float32)
          + jnp.dot(h_b[...], wr_r[...], preferred_element_type=jnp.float32)
          + bias_r[...])
    xn_o[...] = xn

    bvec = b3_b[0, 0, :]
    onehot_t = (lax.broadcasted_iota(jnp.int32, (G, R), 0)
                == bvec[None, :]).astype(jnp.float32)
    s1_s[...] += lax.dot_general(onehot_t, xn, (((1,), (0,)), ((), ())),
                                 preferred_element_type=jnp.float32)
    s2_s[...] += lax.dot_general(onehot_t, xn * xn, (((1,), (0,)), ((), ())),
                                 preferred_element_type=jnp.float32)
    cnt_s[...] += jnp.broadcast_to(jnp.sum(onehot_t, axis=1)[:, None], (G, D))

    @pl.when(i == NB - 1)
    def _fin():
        s1_o[...] = s1_s[...]
        s2_o[...] = s2_s[...]
        cnt_o[...] = cnt_s[...]


def _mm_stats(h, agg, ii, io, b3, wl, wr, bias):
    blk = lambda i: (i, 0)
    rep = lambda i: (0, 0)
    fwd = lambda i: (0, i, 0)
    bwd = lambda i: (1, i, 0)
    return pl.pallas_call(
        _mm_stats_body,
        grid=(NB,),
        in_specs=[
            pl.BlockSpec((R, D), blk),
            pl.BlockSpec((1, R, D), fwd),
            pl.BlockSpec((1, R, D), bwd),
            pl.BlockSpec((R, D), blk),
            pl.BlockSpec((R, D), blk),
            pl.BlockSpec((1, 1, R), lambda i: (i, 0, 0)),
            pl.BlockSpec((D, D), rep),
            pl.BlockSpec((D, D), rep),
            pl.BlockSpec((1, D), rep),
        ],
        out_specs=[
            pl.BlockSpec((R, D), blk),
            pl.BlockSpec((G, D), rep),
            pl.BlockSpec((G, D), rep),
            pl.BlockSpec((G, D), rep),
        ],
        out_shape=[
            jax.ShapeDtypeStruct((N, D), jnp.float32),
            jax.ShapeDtypeStruct((G, D), jnp.float32),
            jax.ShapeDtypeStruct((G, D), jnp.float32),
            jax.ShapeDtypeStruct((G, D), jnp.float32),
        ],
        scratch_shapes=[pltpu.VMEM((G, D), jnp.float32)] * 3,
    )(h, agg, agg, ii, io, b3, wl, wr, bias)


def _norm_body(residual, pool, xn_b, hp_b, b3_b, s1_r, s2_r, cnt_r, w_r, bb_r,
               a_r, out_o, *scratch):
    i = pl.program_id(0)
    cg = jnp.maximum(cnt_r[...], 1.0)
    mean = s1_r[...] / cg
    alpha = a_r[...]
    var = s2_r[...] / cg - (2.0 * alpha - alpha * alpha) * mean * mean
    inv_std = lax.rsqrt(var + EPS)

    bvec = b3_b[0, 0, :]
    onehot_t = (lax.broadcasted_iota(jnp.int32, (G, R), 0)
                == bvec[None, :]).astype(jnp.float32)
    mean_b = lax.dot_general(onehot_t, mean, (((0,), (0,)), ((), ())),
                             preferred_element_type=jnp.float32)
    istd_b = lax.dot_general(onehot_t, inv_std, (((0,), (0,)), ((), ())),
                             preferred_element_type=jnp.float32)
    y = w_r[...] * (xn_b[...] - alpha * mean_b) * istd_b + bb_r[...]
    y = jnp.where(y >= 0.0, y, NEG_SLOPE * y)
    if residual:
        y = y + hp_b[...]

    if pool:
        pool_s = scratch[0]

        @pl.when(i == 0)
        def _init():
            pool_s[...] = jnp.zeros_like(pool_s)

        pool_s[...] += lax.dot_general(onehot_t, y, (((1,), (0,)), ((), ())),
                                       preferred_element_type=jnp.float32)

        @pl.when(i == NB - 1)
        def _fin():
            out_o[...] = pool_s[...] / cg
    else:
        out_o[...] = y


def _norm_apply(xn, hprev, b3, s1, s2, cnt, w, bb, a, residual, pool):
    blk = lambda i: (i, 0)
    rep = lambda i: (0, 0)
    if pool:
        out_spec = pl.BlockSpec((G, D), rep)
        out_shape = jax.ShapeDtypeStruct((G, D), jnp.float32)
        scratch = [pltpu.VMEM((G, D), jnp.float32)]
    else:
        out_spec = pl.BlockSpec((R, D), blk)
        out_shape = jax.ShapeDtypeStruct((N, D), jnp.float32)
        scratch = []
    return pl.pallas_call(
        functools.partial(_norm_body, residual, pool),
        grid=(NB,),
        in_specs=[
            pl.BlockSpec((R, D), blk),
            pl.BlockSpec((R, D), blk),
            pl.BlockSpec((1, 1, R), lambda i: (i, 0, 0)),
            pl.BlockSpec((G, D), rep),
            pl.BlockSpec((G, D), rep),
            pl.BlockSpec((G, D), rep),
            pl.BlockSpec((1, D), rep),
            pl.BlockSpec((1, D), rep),
            pl.BlockSpec((1, D), rep),
        ],
        out_specs=out_spec,
        out_shape=out_shape,
        scratch_shapes=scratch,
    )(xn, hprev, b3, s1, s2, cnt, w, bb, a)


def kernel(x, edge_index, batch, Wl, Wr, b, gn_w, gn_b, gn_a):
    f32 = jnp.float32
    src = edge_index[0].astype(jnp.int32)
    dst = edge_index[1].astype(jnp.int32)
    pad_e = NS * EPADT - E
    srcp = jnp.concatenate(
        [src, jnp.full((pad_e,), TRASH, jnp.int32)]).reshape(NS, NCHUNK, CH)
    dstp = jnp.concatenate(
        [dst, jnp.full((pad_e,), TRASH, jnp.int32)]).reshape(NS, NCHUNK, CH)
    edges = jnp.stack([srcp, dstp], axis=2)
    edges = jnp.concatenate(
        [edges, jnp.full((NS, NCPAD - NCHUNK, 2, CH), TRASH, jnp.int32)],
        axis=1)

    zeros = jnp.zeros((STRIPE, D), f32)

    batch3 = batch.astype(jnp.int32).reshape(NB, 1, R)

    degf = _deg_kernel(edges, zeros, jnp.ones((CH, D), f32))
    ii = 1.0 / jnp.maximum(degf[0, :N], 1.0)
    io = 1.0 / jnp.maximum(degf[1, :N], 1.0)

    h = x
    out = None
    for i in range(L):
        hp = jnp.pad(h, ((0, NPAD - N), (0, 0)))
        agg = _agg_kernel(hp, edges, zeros)
        xn, s1, s2, cnt = _mm_stats(h, agg, ii, io, batch3,
                                    Wl[i], Wr[i], b[i].reshape(1, D))
        last = i == L - 1
        res = i >= RESIDUAL_START
        nxt = _norm_apply(xn, h, batch3, s1, s2, cnt,
                          gn_w[i].reshape(1, D), gn_b[i].reshape(1, D),
                          gn_a[i].reshape(1, D), residual=res, pool=last)
        if last:
            out = nxt
        else:
            h = nxt
    return out


# speedup vs baseline: 5.7329x; 1.0372x over previous
import functools

import jax
import jax.numpy as jnp
from jax import lax
from jax.experimental import pallas as pl
from jax.experimental.pallas import tpu as pltpu
from jax.experimental.pallas import tpu_sc as plsc

N = 10000
E = 320000
D = 128
L = 3
G = 8
NEG_SLOPE = 0.01
EPS = 1e-5
RESIDUAL_START = 1

NC = 2
NS = 16
STRIPE = 632
NPAD = NS * STRIPE
TRASH = N
CH = 112
NCHUNK = 180
NCPAD = NCHUNK + 3
EPADT = NCHUNK * CH
R = 1000
NB = N // R

_mesh = plsc.VectorSubcoreMesh(
    core_axis_name="c", subcore_axis_name="s", num_cores=NC, num_subcores=NS)


HALF = D // 2


@functools.partial(
    pl.kernel,
    out_type=jax.ShapeDtypeStruct((NC, 2, NPAD, HALF), jnp.float32),
    mesh=_mesh,
    scratch_types=[
        pltpu.VMEM_SHARED((NPAD, HALF), jnp.float32),
        pltpu.VMEM_SHARED((NPAD, HALF), jnp.float32),
        pltpu.VMEM((2, 3, 2, CH), jnp.int32),
        pltpu.VMEM((3, CH, HALF), jnp.float32),
        pltpu.SemaphoreType.DMA,
        pltpu.SemaphoreType.DMA,
        pltpu.SemaphoreType.DMA,
        pltpu.SemaphoreType.DMA,
        pltpu.SemaphoreType.DMA,
        pltpu.SemaphoreType.DMA,
    ],
)
def _agg_kernel(h_hbm, edges_hbm, zeros_hbm, agg_hbm,
                h_s, acc_s, pblk_v, buf_v, g0, g1, g2, s0, s1, s2):
    c = lax.axis_index("c")
    s = lax.axis_index("s")
    rows = pl.ds(s * STRIPE, STRIPE)
    gsems = (g0, g1, g2)
    ssems = (s0, s1, s2)

    for ph in range(2):
        pltpu.sync_copy(h_hbm.at[ph, rows], h_s.at[rows])
        pltpu.sync_copy(zeros_hbm, acc_s.at[rows])
        plsc.subcore_barrier()

        pltpu.sync_copy(edges_hbm.at[s, pl.ds(0, 3)], pblk_v.at[1])
        for k in range(3):
            pltpu.async_copy(h_s.at[pblk_v.at[1, k, c]], buf_v.at[k],
                             gsems[k])

        def loop(g, carry):
            b = 3 * g
            p = lax.rem(g, 2)
            pltpu.sync_copy(edges_hbm.at[s, pl.ds(b + 3, 3)], pblk_v.at[p])
            for k in range(3):
                pltpu.make_async_copy(
                    h_s.at[pblk_v.at[1 - p, k, c]], buf_v.at[k],
                    gsems[k]).wait()
                pltpu.async_copy(buf_v.at[k],
                                 acc_s.at[pblk_v.at[1 - p, k, 1 - c]],
                                 ssems[k], add=True)
            for k in range(3):
                ck = b + k

                @pl.when(ck + 3 < NCHUNK)
                def _():
                    pltpu.make_async_copy(
                        buf_v.at[k], acc_s.at[pblk_v.at[1 - p, k, 1 - c]],
                        ssems[k]).wait()
                    pltpu.async_copy(h_s.at[pblk_v.at[p, k, c]], buf_v.at[k],
                                     gsems[k])
            return carry

        lax.fori_loop(0, NCHUNK // 3, loop, 0)
        for k in range(3):
            pltpu.make_async_copy(
                buf_v.at[k], acc_s.at[pblk_v.at[0, k, 1]], ssems[k]).wait()
        plsc.subcore_barrier()
        pltpu.sync_copy(acc_s.at[rows], agg_hbm.at[c, ph, rows])


@functools.partial(
    pl.kernel,
    out_type=jax.ShapeDtypeStruct((NC, NPAD, D), jnp.float32),
    mesh=_mesh,
    scratch_types=[
        pltpu.VMEM_SHARED((NPAD, D), jnp.float32),
        pltpu.VMEM((2, 3, 2, CH), jnp.int32),
        pltpu.VMEM((CH, D), jnp.float32),
        pltpu.SemaphoreType.DMA,
        pltpu.SemaphoreType.DMA,
        pltpu.SemaphoreType.DMA,
    ],
)
def _deg_kernel(edges_hbm, zeros_hbm, ones_hbm, deg_hbm,
                dacc_s, pblk_v, ones_v, s0, s1, s2):
    c = lax.axis_index("c")
    s = lax.axis_index("s")
    rows = pl.ds(s * STRIPE, STRIPE)
    ssems = (s0, s1, s2)
    pltpu.sync_copy(zeros_hbm, dacc_s.at[rows])
    pltpu.sync_copy(ones_hbm, ones_v)
    plsc.subcore_barrier()

    pltpu.sync_copy(edges_hbm.at[s, pl.ds(0, 3)], pblk_v.at[1])
    for k in range(3):
        pltpu.async_copy(ones_v, dacc_s.at[pblk_v.at[1, k, 1 - c]], ssems[k],
                         add=True)

    def loop(g, carry):
        b = 3 * g
        p = lax.rem(g, 2)
        pltpu.sync_copy(edges_hbm.at[s, pl.ds(b + 3, 3)], pblk_v.at[p])
        for k in range(3):
            ck = b + k
            pltpu.make_async_copy(
                ones_v, dacc_s.at[pblk_v.at[1 - p, k, 1 - c]],
                ssems[k]).wait()

            @pl.when(ck + 3 < NCHUNK)
            def _():
                pltpu.async_copy(ones_v, dacc_s.at[pblk_v.at[p, k, 1 - c]],
                                 ssems[k], add=True)
        return carry

    lax.fori_loop(0, NCHUNK // 3, loop, 0)
    plsc.subcore_barrier()
    pltpu.sync_copy(dacc_s.at[rows], deg_hbm.at[c, rows])


def _mm_stats_body(h_b, af_b, ab_b, ii_b, io_b, b3_b, wl_r, wr_r, bias_r,
                   xn_o, s1_o, s2_o, cnt_o, s1_s, s2_s, cnt_s):
    i = pl.program_id(0)

    @pl.when(i == 0)
    def _init():
        s1_s[...] = jnp.zeros_like(s1_s)
        s2_s[...] = jnp.zeros_like(s2_s)
        cnt_s[...] = jnp.zeros_like(cnt_s)

    m = 0.5 * (af_b[...] * ii_b[...] + ab_b[...] * io_b[...])
    xn = (jnp.dot(m, wl_r[...], preferred_element_type=jnp.float32)
          + jnp.dot(h_b[...], wr_r[...], preferred_element_type=jnp.float32)
          + bias_r[...])
    xn_o[...] = xn

    bvec = b3_b[0, 0, :]
    onehot_t = (lax.broadcasted_iota(jnp.int32, (G, R), 0)
                == bvec[None, :]).astype(jnp.float32)
    s1_s[...] += lax.dot_general(onehot_t, xn, (((1,), (0,)), ((), ())),
                                 preferred_element_type=jnp.float32)
    s2_s[...] += lax.dot_general(onehot_t, xn * xn, (((1,), (0,)), ((), ())),
                                 preferred_element_type=jnp.float32)
    cnt_s[...] += jnp.broadcast_to(jnp.sum(onehot_t, axis=1)[:, None], (G, D))

    @pl.when(i == NB - 1)
    def _fin():
        s1_o[...] = s1_s[...]
        s2_o[...] = s2_s[...]
        cnt_o[...] = cnt_s[...]


def _mm_stats(h, aggf, aggb, ii, io, b3, wl, wr, bias):
    blk = lambda i: (i, 0)
    rep = lambda i: (0, 0)
    return pl.pallas_call(
        _mm_stats_body,
        grid=(NB,),
        in_specs=[
            pl.BlockSpec((R, D), blk),
            pl.BlockSpec((R, D), blk),
            pl.BlockSpec((R, D), blk),
            pl.BlockSpec((R, D), blk),
            pl.BlockSpec((R, D), blk),
            pl.BlockSpec((1, 1, R), lambda i: (i, 0, 0)),
            pl.BlockSpec((D, D), rep),
            pl.BlockSpec((D, D), rep),
            pl.BlockSpec((1, D), rep),
        ],
        out_specs=[
            pl.BlockSpec((R, D), blk),
            pl.BlockSpec((G, D), rep),
            pl.BlockSpec((G, D), rep),
            pl.BlockSpec((G, D), rep),
        ],
        out_shape=[
            jax.ShapeDtypeStruct((N, D), jnp.float32),
            jax.ShapeDtypeStruct((G, D), jnp.float32),
            jax.ShapeDtypeStruct((G, D), jnp.float32),
            jax.ShapeDtypeStruct((G, D), jnp.float32),
        ],
        scratch_shapes=[pltpu.VMEM((G, D), jnp.float32)] * 3,
    )(h, aggf, aggb, ii, io, b3, wl, wr, bias)


def _norm_body(residual, pool, xn_b, hp_b, b3_b, s1_r, s2_r, cnt_r, w_r, bb_r,
               a_r, out_o, *scratch):
    i = pl.program_id(0)
    cg = jnp.maximum(cnt_r[...], 1.0)
    mean = s1_r[...] / cg
    alpha = a_r[...]
    var = s2_r[...] / cg - (2.0 * alpha - alpha * alpha) * mean * mean
    inv_std = lax.rsqrt(var + EPS)

    bvec = b3_b[0, 0, :]
    onehot_t = (lax.broadcasted_iota(jnp.int32, (G, R), 0)
                == bvec[None, :]).astype(jnp.float32)
    mean_b = lax.dot_general(onehot_t, mean, (((0,), (0,)), ((), ())),
                             preferred_element_type=jnp.float32)
    istd_b = lax.dot_general(onehot_t, inv_std, (((0,), (0,)), ((), ())),
                             preferred_element_type=jnp.float32)
    y = w_r[...] * (xn_b[...] - alpha * mean_b) * istd_b + bb_r[...]
    y = jnp.where(y >= 0.0, y, NEG_SLOPE * y)
    if residual:
        y = y + hp_b[...]

    if pool:
        pool_s = scratch[0]

        @pl.when(i == 0)
        def _init():
            pool_s[...] = jnp.zeros_like(pool_s)

        pool_s[...] += lax.dot_general(onehot_t, y, (((1,), (0,)), ((), ())),
                                       preferred_element_type=jnp.float32)

        @pl.when(i == NB - 1)
        def _fin():
            out_o[...] = pool_s[...] / cg
    else:
        out_o[...] = y


def _norm_apply(xn, hprev, b3, s1, s2, cnt, w, bb, a, residual, pool):
    blk = lambda i: (i, 0)
    rep = lambda i: (0, 0)
    if pool:
        out_spec = pl.BlockSpec((G, D), rep)
        out_shape = jax.ShapeDtypeStruct((G, D), jnp.float32)
        scratch = [pltpu.VMEM((G, D), jnp.float32)]
    else:
        out_spec = pl.BlockSpec((R, D), blk)
        out_shape = jax.ShapeDtypeStruct((N, D), jnp.float32)
        scratch = []
    return pl.pallas_call(
        functools.partial(_norm_body, residual, pool),
        grid=(NB,),
        in_specs=[
            pl.BlockSpec((R, D), blk),
            pl.BlockSpec((R, D), blk),
            pl.BlockSpec((1, 1, R), lambda i: (i, 0, 0)),
            pl.BlockSpec((G, D), rep),
            pl.BlockSpec((G, D), rep),
            pl.BlockSpec((G, D), rep),
            pl.BlockSpec((1, D), rep),
            pl.BlockSpec((1, D), rep),
            pl.BlockSpec((1, D), rep),
        ],
        out_specs=out_spec,
        out_shape=out_shape,
        scratch_shapes=scratch,
    )(xn, hprev, b3, s1, s2, cnt, w, bb, a)


def kernel(x, edge_index, batch, Wl, Wr, b, gn_w, gn_b, gn_a):
    f32 = jnp.float32
    src = edge_index[0].astype(jnp.int32)
    dst = edge_index[1].astype(jnp.int32)
    pad_e = NS * EPADT - E
    srcp = jnp.concatenate(
        [src, jnp.full((pad_e,), TRASH, jnp.int32)]).reshape(NS, NCHUNK, CH)
    dstp = jnp.concatenate(
        [dst, jnp.full((pad_e,), TRASH, jnp.int32)]).reshape(NS, NCHUNK, CH)
    edges = jnp.stack([srcp, dstp], axis=2)
    edges = jnp.concatenate(
        [edges, jnp.full((NS, NCPAD - NCHUNK, 2, CH), TRASH, jnp.int32)],
        axis=1)

    zeros = jnp.zeros((STRIPE, D), f32)
    zeros_h = jnp.zeros((STRIPE, HALF), f32)

    batch3 = batch.astype(jnp.int32).reshape(NB, 1, R)

    degf = _deg_kernel(edges, zeros, jnp.ones((CH, D), f32))
    ii = 1.0 / jnp.maximum(degf[0, :N], 1.0)
    io = 1.0 / jnp.maximum(degf[1, :N], 1.0)

    h = x
    out = None
    for i in range(L):
        hp = jnp.pad(h, ((0, NPAD - N), (0, 0)))
        hh = hp.reshape(NPAD, 2, HALF).transpose(1, 0, 2)
        agg = _agg_kernel(hh, edges, zeros_h)
        aggf = agg[0].transpose(1, 0, 2).reshape(NPAD, D)[:N]
        aggb = agg[1].transpose(1, 0, 2).reshape(NPAD, D)[:N]
        xn, s1, s2, cnt = _mm_stats(h, aggf, aggb, ii, io, batch3,
                                    Wl[i], Wr[i], b[i].reshape(1, D))
        last = i == L - 1
        res = i >= RESIDUAL_START
        nxt = _norm_apply(xn, h, batch3, s1, s2, cnt,
                          gn_w[i].reshape(1, D), gn_b[i].reshape(1, D),
                          gn_a[i].reshape(1, D), residual=res, pool=last)
        if last:
            out = nxt
        else:
            h = nxt
    return out
